# baseline (speedup 1.0000x reference)
import functools

import jax
import jax.numpy as jnp
from jax import lax
from jax.experimental import pallas as pl
from jax.experimental.pallas import tpu as pltpu
from jax.experimental.pallas import tpu_sc as plsc


@functools.lru_cache(maxsize=None)
def _make_gather(V, D, B):
    info = plsc.get_sparse_core_info()
    nc, ns = info.num_cores, info.num_subcores
    nw = nc * ns
    assert B % nw == 0 and (B // nw) % 8 == 0
    b_per_w = B // nw
    mesh = plsc.VectorSubcoreMesh(core_axis_name="c", subcore_axis_name="s")

    @functools.partial(
        pl.kernel, mesh=mesh,
        out_type=jax.ShapeDtypeStruct((B, D), jnp.float32),
        compiler_params=pltpu.CompilerParams(use_tc_tiling_on_sc=False),
        scratch_types=[
            pltpu.VMEM((b_per_w,), jnp.int32),
            pltpu.VMEM((b_per_w, D), jnp.float32),
            pltpu.SemaphoreType.DMA,
        ],
    )
    def gather_kernel(table_hbm, idx_hbm, out_hbm, idx_v, rows_v, sem):
        wid = lax.axis_index("s") * nc + lax.axis_index("c")
        base = wid * b_per_w
        pltpu.sync_copy(idx_hbm.at[pl.ds(base, b_per_w)], idx_v)
        pltpu.async_copy(table_hbm.at[idx_v], rows_v, sem).wait()
        pltpu.sync_copy(rows_v, out_hbm.at[pl.ds(base, b_per_w)])

    return gather_kernel


def _dense_body(x_ref, enc_ref, dec_ref, bias_ref, out_ref, loss_ref):
    x = x_ref[:]
    enc = enc_ref[:]
    dec = dec_ref[:]
    h = lax.dot_general(x, enc, (((1,), (1,)), ((), ())),
                        preferred_element_type=jnp.float32)
    binary = (h >= 0).astype(jnp.float32)
    y = lax.dot_general(binary, dec, (((1,), (1,)), ((), ())),
                        preferred_element_type=jnp.float32)
    out_ref[:] = y + bias_ref[:]
    corr = lax.dot_general(dec, enc, (((1,), (0,)), ((), ())),
                           preferred_element_type=jnp.float32)
    n = corr.shape[0]
    eye = (lax.broadcasted_iota(jnp.int32, (n, n), 0)
           == lax.broadcasted_iota(jnp.int32, (n, n), 1)).astype(jnp.float32)
    diff = corr - eye
    loss_ref[0, 0] = jnp.sqrt(jnp.sum(diff * diff))


@functools.lru_cache(maxsize=None)
def _make_dense(B, D, H, interpret=False):
    return pl.pallas_call(
        _dense_body,
        out_shape=(jax.ShapeDtypeStruct((B, D), jnp.float32),
                   jax.ShapeDtypeStruct((1, 1), jnp.float32)),
        in_specs=[pl.BlockSpec(memory_space=pltpu.VMEM)] * 4,
        out_specs=(pl.BlockSpec(memory_space=pltpu.VMEM),
                   pl.BlockSpec(memory_space=pltpu.SMEM)),
        interpret=interpret,
    )


def kernel(input, emb_table, enc_w, dec_w, dec_b):
    idx = input.astype(jnp.int32)
    (B,) = idx.shape
    V, D = emb_table.shape
    H = enc_w.shape[0]
    in_embed = _make_gather(V, D, B)(emb_table, idx)
    out_embed, loss = _make_dense(B, D, H)(
        in_embed, enc_w, dec_w, dec_b.reshape(1, D))
    return in_embed, out_embed, loss.reshape(())
